# baseline (speedup 1.0000x reference)
import dataclasses
import functools

import numpy as np
import jax
import jax.numpy as jnp
from jax import lax
from jax.experimental import pallas as pl
from jax.experimental.pallas import tpu as pltpu
from jax.experimental.pallas import tpu_sc as plsc

DIM = 2048
DEPTH = 7
PAR = 16
NN = 255
NP = 256
NWORKERS = 32
LANES = 16

_BT1 = 1024
_BT2 = 512
_CHUNKS = 4


def _k1_body(x_ref, w_ref, b_ref, acts_ref):
    logits = lax.dot_general(
        x_ref[...], w_ref[0], (((1,), (1,)), ((), ())),
        preferred_element_type=jnp.float32,
    ) + b_ref[0]
    acts_ref[0] = logits * jax.nn.sigmoid(logits)


def _k3_body(acts_ref, w_ref, out_ref):
    acc = None
    for p in range(PAR):
        acts = acts_ref[p].astype(jnp.bfloat16)
        prod = jnp.dot(acts, w_ref[p], preferred_element_type=jnp.float32)
        acc = prod if acc is None else acc + prod
    out_ref[...] = acc


_ROWS = 64
_SUBS = _ROWS // LANES


def _sc_route(acts_hbm, out_hbm, abuf0, abuf1, obuf0, obuf1, curs0, curs1,
              sem_i0, sem_i1, sem_o0, sem_o1):
    wid = lax.axis_index("s") * 2 + lax.axis_index("c")
    rows_total = acts_hbm.shape[0]
    rows_per_w = rows_total // NWORKERS
    nblocks = rows_per_w // _ROWS
    base0 = wid * rows_per_w

    iota = lax.iota(jnp.int32, LANES)
    zeros16f = jnp.zeros((LANES,), jnp.float32)
    zeros16i = jnp.zeros((LANES,), jnp.int32)

    for ob in (obuf0, obuf1):
        @pl.loop(0, _ROWS)
        def _(r):
            @pl.loop(0, NP, step=LANES)
            def _(c):
                ob[r, pl.ds(c, LANES)] = zeros16f
    for cb in (curs0, curs1):
        @pl.loop(0, _SUBS * (DEPTH + 1))
        def _(k):
            cb[k] = zeros16i

    def start_in(blk, ab, sem):
        pltpu.make_async_copy(
            acts_hbm.at[pl.ds(base0 + blk * _ROWS, _ROWS)], ab, sem).start()

    def wait_in(ab, sem):
        pltpu.make_async_copy(
            acts_hbm.at[pl.ds(base0, _ROWS)], ab, sem).wait()

    def start_out(blk, ob, sem):
        pltpu.make_async_copy(
            ob, out_hbm.at[pl.ds(base0 + blk * _ROWS, _ROWS)], sem).start()

    def wait_out(ob, sem):
        pltpu.make_async_copy(
            ob, out_hbm.at[pl.ds(base0, _ROWS)], sem).wait()

    def process(ab, ob, cb):
        for s in range(_SUBS):
            rows = iota + (s * LANES)
            for d in range(DEPTH + 1):
                prev = cb[s * (DEPTH + 1) + d]
                plsc.store_scatter(ob, [rows, prev], zeros16f)
            cur = zeros16i
            val = plsc.load_gather(ab, [rows, cur])
            plsc.store_scatter(ob, [rows, cur], val)
            cb[s * (DEPTH + 1)] = cur
            for d in range(DEPTH):
                move = jnp.where(val > 0, 1, 0).astype(jnp.int32)
                cur = 2 * cur + 1 + move
                val = plsc.load_gather(ab, [rows, cur])
                plsc.store_scatter(ob, [rows, cur], val)
                cb[s * (DEPTH + 1) + d + 1] = cur

    start_in(0, abuf0, sem_i0)
    start_in(1, abuf1, sem_i1)

    @pl.loop(0, nblocks, step=2)
    def _(b):
        for r, (ab, ob, cb, si, so) in enumerate((
                (abuf0, obuf0, curs0, sem_i0, sem_o0),
                (abuf1, obuf1, curs1, sem_i1, sem_o1))):
            blk = b + r
            wait_in(ab, si)

            @pl.when(blk >= 2)
            def _():
                wait_out(ob, so)

            process(ab, ob, cb)
            start_out(blk, ob, so)

            @pl.when(blk + 2 < nblocks)
            def _():
                start_in(blk + 2, ab, si)

    wait_out(obuf0, sem_o0)
    wait_out(obuf1, sem_o1)


def kernel(oldx, W_in, b_in, W_out):
    x = oldx.reshape(-1, DIM)
    B = x.shape[0]

    Wr = jnp.pad(W_in.reshape(PAR, NN, DIM),
                 ((0, 0), (0, 1), (0, 0))).astype(jnp.bfloat16)
    br = jnp.pad(b_in.reshape(PAR, 1, NN), ((0, 0), (0, 0), (0, 1)))
    Wo = jnp.pad(W_out.T.reshape(PAR, NN, DIM),
                 ((0, 0), (0, 1), (0, 0))).astype(jnp.bfloat16)
    x16 = x.astype(jnp.bfloat16)

    mesh = plsc.VectorSubcoreMesh(core_axis_name="c", subcore_axis_name="s")
    sc_params = pltpu.CompilerParams()
    if "needs_layout_passes" in pltpu.CompilerParams.__dataclass_fields__:
        sc_params = dataclasses.replace(sc_params, needs_layout_passes=False)

    nchunks = _CHUNKS if B % (_CHUNKS * _BT1) == 0 else 1
    bc = B // nchunks
    bt1 = min(_BT1, bc)
    tpc1 = bc // bt1
    bt2 = min(_BT2, bc)
    tpc2 = bc // bt2

    outs = []
    for c in range(nchunks):
        acts_c = pl.pallas_call(
            _k1_body,
            grid=(tpc1, PAR),
            in_specs=[
                pl.BlockSpec((bt1, DIM), lambda i, p, c=c: (c * tpc1 + i, 0)),
                pl.BlockSpec((1, NP, DIM), lambda i, p: (p, 0, 0)),
                pl.BlockSpec((1, 1, NP), lambda i, p: (p, 0, 0)),
            ],
            out_specs=pl.BlockSpec((1, bt1, NP), lambda i, p: (p, i, 0)),
            out_shape=jax.ShapeDtypeStruct((PAR, bc, NP), jnp.float32),
        )(x16, Wr, br)

        masked_flat = pl.kernel(
            _sc_route,
            out_type=jax.ShapeDtypeStruct((PAR * bc, NP), jnp.float32),
            mesh=mesh,
            compiler_params=sc_params,
            scratch_types=[
                pltpu.VMEM((_ROWS, NP), jnp.float32),
                pltpu.VMEM((_ROWS, NP), jnp.float32),
                pltpu.VMEM((_ROWS, NP), jnp.float32),
                pltpu.VMEM((_ROWS, NP), jnp.float32),
                pltpu.VMEM((_SUBS * (DEPTH + 1), LANES), jnp.int32),
                pltpu.VMEM((_SUBS * (DEPTH + 1), LANES), jnp.int32),
                pltpu.SemaphoreType.DMA,
                pltpu.SemaphoreType.DMA,
                pltpu.SemaphoreType.DMA,
                pltpu.SemaphoreType.DMA,
            ],
        )(acts_c.reshape(PAR * bc, NP))
        masked = masked_flat.reshape(PAR, bc, NP)

        out_c = pl.pallas_call(
            _k3_body,
            grid=(tpc2,),
            in_specs=[
                pl.BlockSpec((PAR, bt2, NP), lambda i: (0, i, 0)),
                pl.BlockSpec((PAR, NP, DIM), lambda i: (0, 0, 0)),
            ],
            out_specs=pl.BlockSpec((bt2, DIM), lambda i: (i, 0)),
            out_shape=jax.ShapeDtypeStruct((bc, DIM), jnp.float32),
        )(masked, Wo)
        outs.append(out_c)

    out = outs[0] if nchunks == 1 else jnp.concatenate(outs, axis=0)
    return out.reshape(oldx.shape)


# speedup vs baseline: 1.0354x; 1.0354x over previous
import dataclasses
import functools

import numpy as np
import jax
import jax.numpy as jnp
from jax import lax
from jax.experimental import pallas as pl
from jax.experimental.pallas import tpu as pltpu
from jax.experimental.pallas import tpu_sc as plsc

DIM = 2048
DEPTH = 7
PAR = 16
NN = 255
NP = 256
NWORKERS = 32
LANES = 16

_BT1 = 1024
_BT2 = 512
_CHUNKS = 2


def _k1_body(x_ref, w_ref, b_ref, acts_ref):
    logits = lax.dot_general(
        x_ref[...], w_ref[0], (((1,), (1,)), ((), ())),
        preferred_element_type=jnp.float32,
    ) + b_ref[0]
    acts_ref[0] = logits * jax.nn.sigmoid(logits)


def _k3_body(acts_ref, w_ref, out_ref):
    acc = None
    for p in range(PAR):
        acts = acts_ref[p].astype(jnp.bfloat16)
        prod = jnp.dot(acts, w_ref[p], preferred_element_type=jnp.float32)
        acc = prod if acc is None else acc + prod
    out_ref[...] = acc


_ROWS = 64
_SUBS = _ROWS // LANES


def _sc_route(acts_hbm, out_hbm, abuf0, abuf1, obuf0, obuf1, curs0, curs1,
              sem_i0, sem_i1, sem_o0, sem_o1):
    wid = lax.axis_index("s") * 2 + lax.axis_index("c")
    rows_total = acts_hbm.shape[0]
    rows_per_w = rows_total // NWORKERS
    nblocks = rows_per_w // _ROWS
    base0 = wid * rows_per_w

    iota = lax.iota(jnp.int32, LANES)
    zeros16f = jnp.zeros((LANES,), jnp.float32)
    zeros16i = jnp.zeros((LANES,), jnp.int32)

    for ob in (obuf0, obuf1):
        @pl.loop(0, _ROWS)
        def _(r):
            @pl.loop(0, NP, step=LANES)
            def _(c):
                ob[r, pl.ds(c, LANES)] = zeros16f
    for cb in (curs0, curs1):
        @pl.loop(0, _SUBS * (DEPTH + 1))
        def _(k):
            cb[k] = zeros16i

    def start_in(blk, ab, sem):
        pltpu.make_async_copy(
            acts_hbm.at[pl.ds(base0 + blk * _ROWS, _ROWS)], ab, sem).start()

    def wait_in(ab, sem):
        pltpu.make_async_copy(
            acts_hbm.at[pl.ds(base0, _ROWS)], ab, sem).wait()

    def start_out(blk, ob, sem):
        pltpu.make_async_copy(
            ob, out_hbm.at[pl.ds(base0 + blk * _ROWS, _ROWS)], sem).start()

    def wait_out(ob, sem):
        pltpu.make_async_copy(
            ob, out_hbm.at[pl.ds(base0, _ROWS)], sem).wait()

    def process(ab, ob, cb):
        for s in range(_SUBS):
            rows = iota + (s * LANES)
            for d in range(DEPTH + 1):
                prev = cb[s * (DEPTH + 1) + d]
                plsc.store_scatter(ob, [rows, prev], zeros16f)
            cur = zeros16i
            val = plsc.load_gather(ab, [rows, cur])
            plsc.store_scatter(ob, [rows, cur], val)
            cb[s * (DEPTH + 1)] = cur
            for d in range(DEPTH):
                move = jnp.where(val > 0, 1, 0).astype(jnp.int32)
                cur = 2 * cur + 1 + move
                val = plsc.load_gather(ab, [rows, cur])
                plsc.store_scatter(ob, [rows, cur], val)
                cb[s * (DEPTH + 1) + d + 1] = cur

    start_in(0, abuf0, sem_i0)
    start_in(1, abuf1, sem_i1)

    @pl.loop(0, nblocks, step=2)
    def _(b):
        for r, (ab, ob, cb, si, so) in enumerate((
                (abuf0, obuf0, curs0, sem_i0, sem_o0),
                (abuf1, obuf1, curs1, sem_i1, sem_o1))):
            blk = b + r
            wait_in(ab, si)

            @pl.when(blk >= 2)
            def _():
                wait_out(ob, so)

            process(ab, ob, cb)
            start_out(blk, ob, so)

            @pl.when(blk + 2 < nblocks)
            def _():
                start_in(blk + 2, ab, si)

    wait_out(obuf0, sem_o0)
    wait_out(obuf1, sem_o1)


def kernel(oldx, W_in, b_in, W_out):
    x = oldx.reshape(-1, DIM)
    B = x.shape[0]

    Wr = jnp.pad(W_in.reshape(PAR, NN, DIM),
                 ((0, 0), (0, 1), (0, 0))).astype(jnp.bfloat16)
    br = jnp.pad(b_in.reshape(PAR, 1, NN), ((0, 0), (0, 0), (0, 1)))
    Wo = jnp.pad(W_out.T.reshape(PAR, NN, DIM),
                 ((0, 0), (0, 1), (0, 0))).astype(jnp.bfloat16)
    x16 = x.astype(jnp.bfloat16)

    mesh = plsc.VectorSubcoreMesh(core_axis_name="c", subcore_axis_name="s")
    sc_params = pltpu.CompilerParams()
    if "needs_layout_passes" in pltpu.CompilerParams.__dataclass_fields__:
        sc_params = dataclasses.replace(sc_params, needs_layout_passes=False)

    nchunks = _CHUNKS if B % (_CHUNKS * _BT1) == 0 else 1
    bc = B // nchunks
    bt1 = min(_BT1, bc)
    tpc1 = bc // bt1
    bt2 = min(_BT2, bc)
    tpc2 = bc // bt2

    outs = []
    for c in range(nchunks):
        acts_c = pl.pallas_call(
            _k1_body,
            grid=(tpc1, PAR),
            in_specs=[
                pl.BlockSpec((bt1, DIM), lambda i, p, c=c: (c * tpc1 + i, 0)),
                pl.BlockSpec((1, NP, DIM), lambda i, p: (p, 0, 0)),
                pl.BlockSpec((1, 1, NP), lambda i, p: (p, 0, 0)),
            ],
            out_specs=pl.BlockSpec((1, bt1, NP), lambda i, p: (p, i, 0)),
            out_shape=jax.ShapeDtypeStruct((PAR, bc, NP), jnp.float32),
        )(x16, Wr, br)

        masked_flat = pl.kernel(
            _sc_route,
            out_type=jax.ShapeDtypeStruct((PAR * bc, NP), jnp.float32),
            mesh=mesh,
            compiler_params=sc_params,
            scratch_types=[
                pltpu.VMEM((_ROWS, NP), jnp.float32),
                pltpu.VMEM((_ROWS, NP), jnp.float32),
                pltpu.VMEM((_ROWS, NP), jnp.float32),
                pltpu.VMEM((_ROWS, NP), jnp.float32),
                pltpu.VMEM((_SUBS * (DEPTH + 1), LANES), jnp.int32),
                pltpu.VMEM((_SUBS * (DEPTH + 1), LANES), jnp.int32),
                pltpu.SemaphoreType.DMA,
                pltpu.SemaphoreType.DMA,
                pltpu.SemaphoreType.DMA,
                pltpu.SemaphoreType.DMA,
            ],
        )(acts_c.reshape(PAR * bc, NP))
        masked = masked_flat.reshape(PAR, bc, NP)

        out_c = pl.pallas_call(
            _k3_body,
            grid=(tpc2,),
            in_specs=[
                pl.BlockSpec((PAR, bt2, NP), lambda i: (0, i, 0)),
                pl.BlockSpec((PAR, NP, DIM), lambda i: (0, 0, 0)),
            ],
            out_specs=pl.BlockSpec((bt2, DIM), lambda i: (i, 0)),
            out_shape=jax.ShapeDtypeStruct((bc, DIM), jnp.float32),
        )(masked, Wo)
        outs.append(out_c)

    out = outs[0] if nchunks == 1 else jnp.concatenate(outs, axis=0)
    return out.reshape(oldx.shape)


# speedup vs baseline: 1.0748x; 1.0380x over previous
import dataclasses
import functools

import numpy as np
import jax
import jax.numpy as jnp
from jax import lax
from jax.experimental import pallas as pl
from jax.experimental.pallas import tpu as pltpu
from jax.experimental.pallas import tpu_sc as plsc

DIM = 2048
DEPTH = 7
PAR = 16
NN = 255
NP = 256
NWORKERS = 32
LANES = 16

_BT1 = 1024
_BT2 = 512
_CHUNKS = 1


def _k1_body(x_ref, w_ref, b_ref, acts_ref):
    logits = lax.dot_general(
        x_ref[...], w_ref[0], (((1,), (1,)), ((), ())),
        preferred_element_type=jnp.float32,
    ) + b_ref[0]
    acts_ref[0] = logits * jax.nn.sigmoid(logits)


def _k3_body(acts_ref, w_ref, out_ref):
    acc = None
    for p in range(PAR):
        acts = acts_ref[p].astype(jnp.bfloat16)
        w = w_ref[:, p * NP:(p + 1) * NP]
        prod = lax.dot_general(acts, w, (((1,), (1,)), ((), ())),
                               preferred_element_type=jnp.float32)
        acc = prod if acc is None else acc + prod
    out_ref[...] = acc


_ROWS = 64
_SUBS = _ROWS // LANES


def _sc_route(acts_hbm, out_hbm, abuf0, abuf1, obuf0, obuf1, curs0, curs1,
              sem_i0, sem_i1, sem_o0, sem_o1):
    wid = lax.axis_index("s") * 2 + lax.axis_index("c")
    rows_total = acts_hbm.shape[0]
    rows_per_w = rows_total // NWORKERS
    nblocks = rows_per_w // _ROWS
    base0 = wid * rows_per_w

    iota = lax.iota(jnp.int32, LANES)
    zeros16f = jnp.zeros((LANES,), jnp.float32)
    zeros16i = jnp.zeros((LANES,), jnp.int32)

    for ob in (obuf0, obuf1):
        @pl.loop(0, _ROWS)
        def _(r):
            @pl.loop(0, NP, step=LANES)
            def _(c):
                ob[r, pl.ds(c, LANES)] = zeros16f
    for cb in (curs0, curs1):
        @pl.loop(0, _SUBS * (DEPTH + 1))
        def _(k):
            cb[k] = zeros16i

    def start_in(blk, ab, sem):
        pltpu.make_async_copy(
            acts_hbm.at[pl.ds(base0 + blk * _ROWS, _ROWS)], ab, sem).start()

    def wait_in(ab, sem):
        pltpu.make_async_copy(
            acts_hbm.at[pl.ds(base0, _ROWS)], ab, sem).wait()

    def start_out(blk, ob, sem):
        pltpu.make_async_copy(
            ob, out_hbm.at[pl.ds(base0 + blk * _ROWS, _ROWS)], sem).start()

    def wait_out(ob, sem):
        pltpu.make_async_copy(
            ob, out_hbm.at[pl.ds(base0, _ROWS)], sem).wait()

    def process(ab, ob, cb):
        for s in range(_SUBS):
            rows = iota + (s * LANES)
            for d in range(DEPTH + 1):
                prev = cb[s * (DEPTH + 1) + d]
                plsc.store_scatter(ob, [rows, prev], zeros16f)
            cur = zeros16i
            val = plsc.load_gather(ab, [rows, cur])
            plsc.store_scatter(ob, [rows, cur], val)
            cb[s * (DEPTH + 1)] = cur
            for d in range(DEPTH):
                move = jnp.where(val > 0, 1, 0).astype(jnp.int32)
                cur = 2 * cur + 1 + move
                val = plsc.load_gather(ab, [rows, cur])
                plsc.store_scatter(ob, [rows, cur], val)
                cb[s * (DEPTH + 1) + d + 1] = cur

    start_in(0, abuf0, sem_i0)
    start_in(1, abuf1, sem_i1)

    @pl.loop(0, nblocks, step=2)
    def _(b):
        for r, (ab, ob, cb, si, so) in enumerate((
                (abuf0, obuf0, curs0, sem_i0, sem_o0),
                (abuf1, obuf1, curs1, sem_i1, sem_o1))):
            blk = b + r
            wait_in(ab, si)

            @pl.when(blk >= 2)
            def _():
                wait_out(ob, so)

            process(ab, ob, cb)
            start_out(blk, ob, so)

            @pl.when(blk + 2 < nblocks)
            def _():
                start_in(blk + 2, ab, si)

    wait_out(obuf0, sem_o0)
    wait_out(obuf1, sem_o1)


def kernel(oldx, W_in, b_in, W_out):
    x = oldx.reshape(-1, DIM)
    B = x.shape[0]

    Wr = jnp.pad(W_in.reshape(PAR, NN, DIM),
                 ((0, 0), (0, 1), (0, 0))).astype(jnp.bfloat16)
    br = jnp.pad(b_in.reshape(PAR, 1, NN), ((0, 0), (0, 0), (0, 1)))
    Wo = jnp.pad(W_out.reshape(DIM, PAR, NN),
                 ((0, 0), (0, 0), (0, 1))).reshape(DIM, PAR * NP)
    Wo = Wo.astype(jnp.bfloat16)
    x16 = x.astype(jnp.bfloat16)

    mesh = plsc.VectorSubcoreMesh(core_axis_name="c", subcore_axis_name="s")
    sc_params = pltpu.CompilerParams()
    if "needs_layout_passes" in pltpu.CompilerParams.__dataclass_fields__:
        sc_params = dataclasses.replace(sc_params, needs_layout_passes=False)

    nchunks = _CHUNKS if B % (_CHUNKS * _BT1) == 0 else 1
    bc = B // nchunks
    bt1 = min(_BT1, bc)
    tpc1 = bc // bt1
    bt2 = min(_BT2, bc)
    tpc2 = bc // bt2

    outs = []
    for c in range(nchunks):
        acts_c = pl.pallas_call(
            _k1_body,
            grid=(tpc1, PAR),
            in_specs=[
                pl.BlockSpec((bt1, DIM), lambda i, p, c=c: (c * tpc1 + i, 0)),
                pl.BlockSpec((1, NP, DIM), lambda i, p: (p, 0, 0)),
                pl.BlockSpec((1, 1, NP), lambda i, p: (p, 0, 0)),
            ],
            out_specs=pl.BlockSpec((1, bt1, NP), lambda i, p: (p, i, 0)),
            out_shape=jax.ShapeDtypeStruct((PAR, bc, NP), jnp.float32),
        )(x16, Wr, br)

        masked_flat = pl.kernel(
            _sc_route,
            out_type=jax.ShapeDtypeStruct((PAR * bc, NP), jnp.float32),
            mesh=mesh,
            compiler_params=sc_params,
            scratch_types=[
                pltpu.VMEM((_ROWS, NP), jnp.float32),
                pltpu.VMEM((_ROWS, NP), jnp.float32),
                pltpu.VMEM((_ROWS, NP), jnp.float32),
                pltpu.VMEM((_ROWS, NP), jnp.float32),
                pltpu.VMEM((_SUBS * (DEPTH + 1), LANES), jnp.int32),
                pltpu.VMEM((_SUBS * (DEPTH + 1), LANES), jnp.int32),
                pltpu.SemaphoreType.DMA,
                pltpu.SemaphoreType.DMA,
                pltpu.SemaphoreType.DMA,
                pltpu.SemaphoreType.DMA,
            ],
        )(acts_c.reshape(PAR * bc, NP))
        masked = masked_flat.reshape(PAR, bc, NP)

        out_c = pl.pallas_call(
            _k3_body,
            grid=(tpc2,),
            in_specs=[
                pl.BlockSpec((PAR, bt2, NP), lambda i: (0, i, 0)),
                pl.BlockSpec((DIM, PAR * NP), lambda i: (0, 0)),
            ],
            out_specs=pl.BlockSpec((bt2, DIM), lambda i: (i, 0)),
            out_shape=jax.ShapeDtypeStruct((bc, DIM), jnp.float32),
        )(masked, Wo)
        outs.append(out_c)

    out = outs[0] if nchunks == 1 else jnp.concatenate(outs, axis=0)
    return out.reshape(oldx.shape)


# speedup vs baseline: 1.1406x; 1.0613x over previous
import dataclasses
import functools

import numpy as np
import jax
import jax.numpy as jnp
from jax import lax
from jax.experimental import pallas as pl
from jax.experimental.pallas import tpu as pltpu
from jax.experimental.pallas import tpu_sc as plsc

DIM = 2048
DEPTH = 7
PAR = 16
NN = 255
NP = 256
NWORKERS = 32
LANES = 16

_BT1 = 2048
_BT2 = 512
_CHUNKS = 1


def _k1_body(x_ref, w_ref, b_ref, acts_ref):
    logits = lax.dot_general(
        x_ref[...], w_ref[0], (((1,), (1,)), ((), ())),
        preferred_element_type=jnp.float32,
    ) + b_ref[0]
    acts_ref[0] = logits * jax.nn.sigmoid(logits)


def _k3_body(acts_ref, w_ref, out_ref):
    acc = None
    for p in range(PAR):
        acts = acts_ref[p].astype(jnp.bfloat16)
        prod = jnp.dot(acts, w_ref[p], preferred_element_type=jnp.float32)
        acc = prod if acc is None else acc + prod
    out_ref[...] = acc


_ROWS = 64
_SUBS = _ROWS // LANES


def _sc_route(acts_hbm, out_hbm, abuf0, abuf1, obuf0, obuf1, curs0, curs1,
              sem_i0, sem_i1, sem_o0, sem_o1):
    wid = lax.axis_index("s") * 2 + lax.axis_index("c")
    rows_total = acts_hbm.shape[0]
    rows_per_w = rows_total // NWORKERS
    nblocks = rows_per_w // _ROWS
    base0 = wid * rows_per_w

    iota = lax.iota(jnp.int32, LANES)
    zeros16f = jnp.zeros((LANES,), jnp.float32)
    zeros16i = jnp.zeros((LANES,), jnp.int32)

    for ob in (obuf0, obuf1):
        @pl.loop(0, _ROWS)
        def _(r):
            @pl.loop(0, NP, step=LANES)
            def _(c):
                ob[r, pl.ds(c, LANES)] = zeros16f
    for cb in (curs0, curs1):
        @pl.loop(0, _SUBS * (DEPTH + 1))
        def _(k):
            cb[k] = zeros16i

    def start_in(blk, ab, sem):
        pltpu.make_async_copy(
            acts_hbm.at[pl.ds(base0 + blk * _ROWS, _ROWS)], ab, sem).start()

    def wait_in(ab, sem):
        pltpu.make_async_copy(
            acts_hbm.at[pl.ds(base0, _ROWS)], ab, sem).wait()

    def start_out(blk, ob, sem):
        pltpu.make_async_copy(
            ob, out_hbm.at[pl.ds(base0 + blk * _ROWS, _ROWS)], sem).start()

    def wait_out(ob, sem):
        pltpu.make_async_copy(
            ob, out_hbm.at[pl.ds(base0, _ROWS)], sem).wait()

    def process(ab, ob, cb):
        for s in range(_SUBS):
            rows = iota + (s * LANES)
            for d in range(DEPTH + 1):
                prev = cb[s * (DEPTH + 1) + d]
                plsc.store_scatter(ob, [rows, prev], zeros16f)
            cur = zeros16i
            val = plsc.load_gather(ab, [rows, cur])
            plsc.store_scatter(ob, [rows, cur], val)
            cb[s * (DEPTH + 1)] = cur
            for d in range(DEPTH):
                move = jnp.where(val > 0, 1, 0).astype(jnp.int32)
                cur = 2 * cur + 1 + move
                val = plsc.load_gather(ab, [rows, cur])
                plsc.store_scatter(ob, [rows, cur], val)
                cb[s * (DEPTH + 1) + d + 1] = cur

    start_in(0, abuf0, sem_i0)
    start_in(1, abuf1, sem_i1)

    @pl.loop(0, nblocks, step=2)
    def _(b):
        for r, (ab, ob, cb, si, so) in enumerate((
                (abuf0, obuf0, curs0, sem_i0, sem_o0),
                (abuf1, obuf1, curs1, sem_i1, sem_o1))):
            blk = b + r
            wait_in(ab, si)

            @pl.when(blk >= 2)
            def _():
                wait_out(ob, so)

            process(ab, ob, cb)
            start_out(blk, ob, so)

            @pl.when(blk + 2 < nblocks)
            def _():
                start_in(blk + 2, ab, si)

    wait_out(obuf0, sem_o0)
    wait_out(obuf1, sem_o1)


def kernel(oldx, W_in, b_in, W_out):
    x = oldx.reshape(-1, DIM)
    B = x.shape[0]

    Wr = jnp.pad(W_in.reshape(PAR, NN, DIM),
                 ((0, 0), (0, 1), (0, 0))).astype(jnp.bfloat16)
    br = jnp.pad(b_in.reshape(PAR, 1, NN), ((0, 0), (0, 0), (0, 1)))
    Wo = jnp.pad(W_out.T.reshape(PAR, NN, DIM),
                 ((0, 0), (0, 1), (0, 0))).astype(jnp.bfloat16)
    x16 = x.astype(jnp.bfloat16)

    mesh = plsc.VectorSubcoreMesh(core_axis_name="c", subcore_axis_name="s")
    sc_params = pltpu.CompilerParams()
    if "needs_layout_passes" in pltpu.CompilerParams.__dataclass_fields__:
        sc_params = dataclasses.replace(sc_params, needs_layout_passes=False)

    nchunks = _CHUNKS if B % (_CHUNKS * _BT1) == 0 else 1
    bc = B // nchunks
    bt1 = min(_BT1, bc)
    tpc1 = bc // bt1
    bt2 = min(_BT2, bc)
    tpc2 = bc // bt2

    outs = []
    for c in range(nchunks):
        acts_c = pl.pallas_call(
            _k1_body,
            grid=(tpc1, PAR),
            in_specs=[
                pl.BlockSpec((bt1, DIM), lambda i, p, c=c: (c * tpc1 + i, 0)),
                pl.BlockSpec((1, NP, DIM), lambda i, p: (p, 0, 0)),
                pl.BlockSpec((1, 1, NP), lambda i, p: (p, 0, 0)),
            ],
            out_specs=pl.BlockSpec((1, bt1, NP), lambda i, p: (p, i, 0)),
            out_shape=jax.ShapeDtypeStruct((PAR, bc, NP), jnp.float32),
        )(x16, Wr, br)

        masked_flat = pl.kernel(
            _sc_route,
            out_type=jax.ShapeDtypeStruct((PAR * bc, NP), jnp.float32),
            mesh=mesh,
            compiler_params=sc_params,
            scratch_types=[
                pltpu.VMEM((_ROWS, NP), jnp.float32),
                pltpu.VMEM((_ROWS, NP), jnp.float32),
                pltpu.VMEM((_ROWS, NP), jnp.float32),
                pltpu.VMEM((_ROWS, NP), jnp.float32),
                pltpu.VMEM((_SUBS * (DEPTH + 1), LANES), jnp.int32),
                pltpu.VMEM((_SUBS * (DEPTH + 1), LANES), jnp.int32),
                pltpu.SemaphoreType.DMA,
                pltpu.SemaphoreType.DMA,
                pltpu.SemaphoreType.DMA,
                pltpu.SemaphoreType.DMA,
            ],
        )(acts_c.reshape(PAR * bc, NP))
        masked = masked_flat.reshape(PAR, bc, NP)

        out_c = pl.pallas_call(
            _k3_body,
            grid=(tpc2,),
            in_specs=[
                pl.BlockSpec((PAR, bt2, NP), lambda i: (0, i, 0)),
                pl.BlockSpec((PAR, NP, DIM), lambda i: (0, 0, 0)),
            ],
            out_specs=pl.BlockSpec((bt2, DIM), lambda i: (i, 0)),
            out_shape=jax.ShapeDtypeStruct((bc, DIM), jnp.float32),
        )(masked, Wo)
        outs.append(out_c)

    out = outs[0] if nchunks == 1 else jnp.concatenate(outs, axis=0)
    return out.reshape(oldx.shape)


# speedup vs baseline: 1.1613x; 1.0181x over previous
import dataclasses
import functools

import numpy as np
import jax
import jax.numpy as jnp
from jax import lax
from jax.experimental import pallas as pl
from jax.experimental.pallas import tpu as pltpu
from jax.experimental.pallas import tpu_sc as plsc

DIM = 2048
DEPTH = 7
PAR = 16
NN = 255
NP = 256
NWORKERS = 32
LANES = 16

_BT1 = 4096
_BT2 = 512
_CHUNKS = 1


def _k1_body(x_ref, w_ref, b_ref, acts_ref):
    logits = lax.dot_general(
        x_ref[...], w_ref[0], (((1,), (1,)), ((), ())),
        preferred_element_type=jnp.float32,
    ) + b_ref[0]
    acts_ref[0] = logits * jax.nn.sigmoid(logits)


def _k3_body(acts_ref, w_ref, out_ref):
    acc = None
    for p in range(PAR):
        acts = acts_ref[p].astype(jnp.bfloat16)
        prod = jnp.dot(acts, w_ref[p], preferred_element_type=jnp.float32)
        acc = prod if acc is None else acc + prod
    out_ref[...] = acc


_ROWS = 64
_SUBS = _ROWS // LANES


def _sc_route(acts_hbm, out_hbm, abuf0, abuf1, obuf0, obuf1, curs0, curs1,
              sem_i0, sem_i1, sem_o0, sem_o1):
    wid = lax.axis_index("s") * 2 + lax.axis_index("c")
    rows_total = acts_hbm.shape[0]
    rows_per_w = rows_total // NWORKERS
    nblocks = rows_per_w // _ROWS
    base0 = wid * rows_per_w

    iota = lax.iota(jnp.int32, LANES)
    zeros16f = jnp.zeros((LANES,), jnp.float32)
    zeros16i = jnp.zeros((LANES,), jnp.int32)

    for ob in (obuf0, obuf1):
        @pl.loop(0, _ROWS)
        def _(r):
            @pl.loop(0, NP, step=LANES)
            def _(c):
                ob[r, pl.ds(c, LANES)] = zeros16f
    for cb in (curs0, curs1):
        @pl.loop(0, _SUBS * (DEPTH + 1))
        def _(k):
            cb[k] = zeros16i

    def start_in(blk, ab, sem):
        pltpu.make_async_copy(
            acts_hbm.at[pl.ds(base0 + blk * _ROWS, _ROWS)], ab, sem).start()

    def wait_in(ab, sem):
        pltpu.make_async_copy(
            acts_hbm.at[pl.ds(base0, _ROWS)], ab, sem).wait()

    def start_out(blk, ob, sem):
        pltpu.make_async_copy(
            ob, out_hbm.at[pl.ds(base0 + blk * _ROWS, _ROWS)], sem).start()

    def wait_out(ob, sem):
        pltpu.make_async_copy(
            ob, out_hbm.at[pl.ds(base0, _ROWS)], sem).wait()

    def process(ab, ob, cb):
        for s in range(_SUBS):
            rows = iota + (s * LANES)
            for d in range(DEPTH + 1):
                prev = cb[s * (DEPTH + 1) + d]
                plsc.store_scatter(ob, [rows, prev], zeros16f)
            cur = zeros16i
            val = plsc.load_gather(ab, [rows, cur])
            plsc.store_scatter(ob, [rows, cur], val)
            cb[s * (DEPTH + 1)] = cur
            for d in range(DEPTH):
                move = jnp.where(val > 0, 1, 0).astype(jnp.int32)
                cur = 2 * cur + 1 + move
                val = plsc.load_gather(ab, [rows, cur])
                plsc.store_scatter(ob, [rows, cur], val)
                cb[s * (DEPTH + 1) + d + 1] = cur

    start_in(0, abuf0, sem_i0)
    start_in(1, abuf1, sem_i1)

    @pl.loop(0, nblocks, step=2)
    def _(b):
        for r, (ab, ob, cb, si, so) in enumerate((
                (abuf0, obuf0, curs0, sem_i0, sem_o0),
                (abuf1, obuf1, curs1, sem_i1, sem_o1))):
            blk = b + r
            wait_in(ab, si)

            @pl.when(blk >= 2)
            def _():
                wait_out(ob, so)

            process(ab, ob, cb)
            start_out(blk, ob, so)

            @pl.when(blk + 2 < nblocks)
            def _():
                start_in(blk + 2, ab, si)

    wait_out(obuf0, sem_o0)
    wait_out(obuf1, sem_o1)


def kernel(oldx, W_in, b_in, W_out):
    x = oldx.reshape(-1, DIM)
    B = x.shape[0]

    Wr = jnp.pad(W_in.reshape(PAR, NN, DIM),
                 ((0, 0), (0, 1), (0, 0))).astype(jnp.bfloat16)
    br = jnp.pad(b_in.reshape(PAR, 1, NN), ((0, 0), (0, 0), (0, 1)))
    Wo = jnp.pad(W_out.T.reshape(PAR, NN, DIM),
                 ((0, 0), (0, 1), (0, 0))).astype(jnp.bfloat16)
    x16 = x.astype(jnp.bfloat16)

    mesh = plsc.VectorSubcoreMesh(core_axis_name="c", subcore_axis_name="s")
    sc_params = pltpu.CompilerParams()
    if "needs_layout_passes" in pltpu.CompilerParams.__dataclass_fields__:
        sc_params = dataclasses.replace(sc_params, needs_layout_passes=False)

    nchunks = _CHUNKS if B % (_CHUNKS * _BT1) == 0 else 1
    bc = B // nchunks
    bt1 = min(_BT1, bc)
    tpc1 = bc // bt1
    bt2 = min(_BT2, bc)
    tpc2 = bc // bt2

    outs = []
    for c in range(nchunks):
        acts_c = pl.pallas_call(
            _k1_body,
            grid=(tpc1, PAR),
            in_specs=[
                pl.BlockSpec((bt1, DIM), lambda i, p, c=c: (c * tpc1 + i, 0)),
                pl.BlockSpec((1, NP, DIM), lambda i, p: (p, 0, 0)),
                pl.BlockSpec((1, 1, NP), lambda i, p: (p, 0, 0)),
            ],
            out_specs=pl.BlockSpec((1, bt1, NP), lambda i, p: (p, i, 0)),
            out_shape=jax.ShapeDtypeStruct((PAR, bc, NP), jnp.float32),
        )(x16, Wr, br)

        masked_flat = pl.kernel(
            _sc_route,
            out_type=jax.ShapeDtypeStruct((PAR * bc, NP), jnp.float32),
            mesh=mesh,
            compiler_params=sc_params,
            scratch_types=[
                pltpu.VMEM((_ROWS, NP), jnp.float32),
                pltpu.VMEM((_ROWS, NP), jnp.float32),
                pltpu.VMEM((_ROWS, NP), jnp.float32),
                pltpu.VMEM((_ROWS, NP), jnp.float32),
                pltpu.VMEM((_SUBS * (DEPTH + 1), LANES), jnp.int32),
                pltpu.VMEM((_SUBS * (DEPTH + 1), LANES), jnp.int32),
                pltpu.SemaphoreType.DMA,
                pltpu.SemaphoreType.DMA,
                pltpu.SemaphoreType.DMA,
                pltpu.SemaphoreType.DMA,
            ],
        )(acts_c.reshape(PAR * bc, NP))
        masked = masked_flat.reshape(PAR, bc, NP)

        out_c = pl.pallas_call(
            _k3_body,
            grid=(tpc2,),
            in_specs=[
                pl.BlockSpec((PAR, bt2, NP), lambda i: (0, i, 0)),
                pl.BlockSpec((PAR, NP, DIM), lambda i: (0, 0, 0)),
            ],
            out_specs=pl.BlockSpec((bt2, DIM), lambda i: (i, 0)),
            out_shape=jax.ShapeDtypeStruct((bc, DIM), jnp.float32),
        )(masked, Wo)
        outs.append(out_c)

    out = outs[0] if nchunks == 1 else jnp.concatenate(outs, axis=0)
    return out.reshape(oldx.shape)


# speedup vs baseline: 1.1618x; 1.0004x over previous
import dataclasses

import jax
import jax.numpy as jnp
from jax import lax
from jax.experimental import pallas as pl
from jax.experimental.pallas import tpu as pltpu
from jax.experimental.pallas import tpu_sc as plsc

DIM = 2048
DEPTH = 7
PAR = 16
NN = 255
NP = 256
NWORKERS = 32
LANES = 16

_BT1 = 4096
_BT2 = 512
_CHUNKS = 1


def _k1_body(x_ref, w_ref, b_ref, acts_ref):
    logits = lax.dot_general(
        x_ref[...], w_ref[0], (((1,), (1,)), ((), ())),
        preferred_element_type=jnp.float32,
    ) + b_ref[0]
    acts_ref[0] = logits * jax.nn.sigmoid(logits)


def _k3_body(acts_ref, w_ref, out_ref):
    acc = None
    for p in range(PAR):
        acts = acts_ref[p].astype(jnp.bfloat16)
        prod = jnp.dot(acts, w_ref[p], preferred_element_type=jnp.float32)
        acc = prod if acc is None else acc + prod
    out_ref[...] = acc


_ROWS = 64
_SUBS = _ROWS // LANES


def _sc_route(acts_hbm, out_hbm, abuf0, abuf1, obuf0, obuf1, curs0, curs1,
              sem_i0, sem_i1, sem_o0, sem_o1):
    wid = lax.axis_index("s") * 2 + lax.axis_index("c")
    rows_total = acts_hbm.shape[0]
    rows_per_w = rows_total // NWORKERS
    nblocks = rows_per_w // _ROWS
    base0 = wid * rows_per_w

    iota = lax.iota(jnp.int32, LANES)
    zeros16f = jnp.zeros((LANES,), jnp.float32)
    zeros16i = jnp.zeros((LANES,), jnp.int32)

    for ob in (obuf0, obuf1):
        @pl.loop(0, _ROWS)
        def _(r):
            @pl.loop(0, NP, step=LANES)
            def _(c):
                ob[r, pl.ds(c, LANES)] = zeros16f
    for cb in (curs0, curs1):
        @pl.loop(0, _SUBS * (DEPTH + 1))
        def _(k):
            cb[k] = zeros16i

    def start_in(blk, ab, sem):
        pltpu.make_async_copy(
            acts_hbm.at[pl.ds(base0 + blk * _ROWS, _ROWS)], ab, sem).start()

    def wait_in(ab, sem):
        pltpu.make_async_copy(
            acts_hbm.at[pl.ds(base0, _ROWS)], ab, sem).wait()

    def start_out(blk, ob, sem):
        pltpu.make_async_copy(
            ob, out_hbm.at[pl.ds(base0 + blk * _ROWS, _ROWS)], sem).start()

    def wait_out(ob, sem):
        pltpu.make_async_copy(
            ob, out_hbm.at[pl.ds(base0, _ROWS)], sem).wait()

    def process(ab, ob, cb):
        for s in range(_SUBS):
            rows = iota + (s * LANES)
            for d in range(DEPTH + 1):
                prev = cb[s * (DEPTH + 1) + d]
                plsc.store_scatter(ob, [rows, prev], zeros16f)
            cur = zeros16i
            val = plsc.load_gather(ab, [rows, cur])
            plsc.store_scatter(ob, [rows, cur], val)
            cb[s * (DEPTH + 1)] = cur
            for d in range(DEPTH):
                move = jnp.where(val > 0, 1, 0).astype(jnp.int32)
                cur = 2 * cur + 1 + move
                val = plsc.load_gather(ab, [rows, cur])
                plsc.store_scatter(ob, [rows, cur], val)
                cb[s * (DEPTH + 1) + d + 1] = cur

    start_in(0, abuf0, sem_i0)
    start_in(1, abuf1, sem_i1)

    @pl.loop(0, nblocks, step=2)
    def _(b):
        for r, (ab, ob, cb, si, so) in enumerate((
                (abuf0, obuf0, curs0, sem_i0, sem_o0),
                (abuf1, obuf1, curs1, sem_i1, sem_o1))):
            blk = b + r
            wait_in(ab, si)

            @pl.when(blk >= 2)
            def _():
                wait_out(ob, so)

            process(ab, ob, cb)
            start_out(blk, ob, so)

            @pl.when(blk + 2 < nblocks)
            def _():
                start_in(blk + 2, ab, si)

    wait_out(obuf0, sem_o0)
    wait_out(obuf1, sem_o1)


def kernel(oldx, W_in, b_in, W_out):
    x = oldx.reshape(-1, DIM)
    B = x.shape[0]

    Wr = jnp.pad(W_in.reshape(PAR, NN, DIM),
                 ((0, 0), (0, 1), (0, 0))).astype(jnp.bfloat16)
    br = jnp.pad(b_in.reshape(PAR, 1, NN), ((0, 0), (0, 0), (0, 1)))
    Wo = jnp.pad(W_out.T.reshape(PAR, NN, DIM),
                 ((0, 0), (0, 1), (0, 0))).astype(jnp.bfloat16)
    x16 = x.astype(jnp.bfloat16)

    mesh = plsc.VectorSubcoreMesh(core_axis_name="c", subcore_axis_name="s")
    sc_params = pltpu.CompilerParams()
    if "needs_layout_passes" in pltpu.CompilerParams.__dataclass_fields__:
        sc_params = dataclasses.replace(sc_params, needs_layout_passes=False)

    nchunks = _CHUNKS if B % (_CHUNKS * _BT1) == 0 else 1
    bc = B // nchunks
    bt1 = min(_BT1, bc)
    tpc1 = bc // bt1
    bt2 = min(_BT2, bc)
    tpc2 = bc // bt2

    outs = []
    for c in range(nchunks):
        acts_c = pl.pallas_call(
            _k1_body,
            grid=(tpc1, PAR),
            in_specs=[
                pl.BlockSpec((bt1, DIM), lambda i, p, c=c: (c * tpc1 + i, 0)),
                pl.BlockSpec((1, NP, DIM), lambda i, p: (p, 0, 0)),
                pl.BlockSpec((1, 1, NP), lambda i, p: (p, 0, 0)),
            ],
            out_specs=pl.BlockSpec((1, bt1, NP), lambda i, p: (p, i, 0)),
            out_shape=jax.ShapeDtypeStruct((PAR, bc, NP), jnp.float32),
        )(x16, Wr, br)

        masked_flat = pl.kernel(
            _sc_route,
            out_type=jax.ShapeDtypeStruct((PAR * bc, NP), jnp.float32),
            mesh=mesh,
            compiler_params=sc_params,
            scratch_types=[
                pltpu.VMEM((_ROWS, NP), jnp.float32),
                pltpu.VMEM((_ROWS, NP), jnp.float32),
                pltpu.VMEM((_ROWS, NP), jnp.float32),
                pltpu.VMEM((_ROWS, NP), jnp.float32),
                pltpu.VMEM((_SUBS * (DEPTH + 1), LANES), jnp.int32),
                pltpu.VMEM((_SUBS * (DEPTH + 1), LANES), jnp.int32),
                pltpu.SemaphoreType.DMA,
                pltpu.SemaphoreType.DMA,
                pltpu.SemaphoreType.DMA,
                pltpu.SemaphoreType.DMA,
            ],
        )(acts_c.reshape(PAR * bc, NP))
        masked = masked_flat.reshape(PAR, bc, NP)

        out_c = pl.pallas_call(
            _k3_body,
            grid=(tpc2,),
            in_specs=[
                pl.BlockSpec((PAR, bt2, NP), lambda i: (0, i, 0)),
                pl.BlockSpec((PAR, NP, DIM), lambda i: (0, 0, 0)),
            ],
            out_specs=pl.BlockSpec((bt2, DIM), lambda i: (i, 0)),
            out_shape=jax.ShapeDtypeStruct((bc, DIM), jnp.float32),
        )(masked, Wo)
        outs.append(out_c)

    out = outs[0] if nchunks == 1 else jnp.concatenate(outs, axis=0)
    return out.reshape(oldx.shape)
